# baseline (speedup 1.0000x reference)
import functools

import jax
import jax.numpy as jnp
from jax import lax
from jax.experimental import pallas as pl
from jax.experimental.pallas import tpu as pltpu
from jax.experimental.pallas import tpu_sc as plsc

N_SRC = 10000
N_DST = 10000
E = 320000
D = 128

NC = 2
NS = 16
L = 16
NW = NC * NS
CHUNK = 128
NROWS = E // CHUNK
STEPS = 80
NROWS_PAD = NW * STEPS
N_PAD = 10240
HR = N_PAD // D
ROWS_PER_TILE = N_PAD // NS
WB = 128

_sc_mesh = plsc.VectorSubcoreMesh(
    core_axis_name="c", subcore_axis_name="s", num_cores=NC, num_subcores=NS)


@functools.partial(
    pl.kernel,
    out_type=(
        jax.ShapeDtypeStruct((NC, N_PAD, D), jnp.float32),
        jax.ShapeDtypeStruct((NC, HR, D), jnp.float32),
    ),
    mesh=_sc_mesh,
    compiler_params=pltpu.CompilerParams(
        use_tc_tiling_on_sc=False, needs_layout_passes=False),
    scratch_types=[
        pltpu.VMEM((2, CHUNK), jnp.int32),
        pltpu.VMEM((2, CHUNK), jnp.int32),
        pltpu.VMEM((2, CHUNK, D), jnp.float32),
        pltpu.VMEM((HR, D), jnp.float32),
        pltpu.VMEM((1, HR), jnp.int32),
        pltpu.VMEM_SHARED((N_PAD, D), jnp.float32),
        pltpu.VMEM_SHARED((HR, D), jnp.float32),
        pltpu.SemaphoreType.DMA,
        pltpu.SemaphoreType.DMA,
        pltpu.SemaphoreType.DMA,
        pltpu.SemaphoreType.DMA,
        pltpu.SemaphoreType.DMA,
        pltpu.SemaphoreType.DMA,
    ],
)
def _sc_aggregate(feat_hbm, src_hbm, dst_hbm, psum_hbm, pdeg_hbm,
                  idx_s, idx_d, rows, hist, idrows,
                  acc_sp, deg_sp, gsem0, gsem1, ssem0, ssem1, isem0, isem1):
    cid = lax.axis_index("c")
    sid = lax.axis_index("s")
    wid = sid * NC + cid
    base = sid * ROWS_PER_TILE
    ones16 = jnp.ones((L,), jnp.float32)

    def _fill_f(i, _):
        rows[0, i // (D // L), pl.ds((i % (D // L)) * L, L)] = (
            jnp.zeros((L,), jnp.float32))
        return 0
    lax.fori_loop(0, WB * (D // L), _fill_f, 0)

    def _fill_h(i, _):
        hist[i // (D // L), pl.ds((i % (D // L)) * L, L)] = (
            jnp.zeros((L,), jnp.float32))
        return 0
    lax.fori_loop(0, HR * (D // L), _fill_h, 0)

    for j in range(HR // L):
        idrows[0, pl.ds(j * L, L)] = lax.iota(jnp.int32, L) + (j * L)

    def _zinit(i, _):
        pltpu.sync_copy(rows.at[0], acc_sp.at[pl.ds(base + i * WB, WB)])
        return 0
    lax.fori_loop(0, ROWS_PER_TILE // WB, _zinit, 0)

    @pl.when(sid == 0)
    def _():
        pltpu.sync_copy(hist, deg_sp)
    plsc.subcore_barrier()

    gsems = (gsem0, gsem1)
    ssems = (ssem0, ssem1)
    isems = (isem0, isem1)

    def _load_idx(b, r):
        pltpu.async_copy(src_hbm.at[r], idx_s.at[b], isems[b])
        pltpu.async_copy(dst_hbm.at[r], idx_d.at[b], isems[b])

    def _wait_idx(b):
        pltpu.make_async_copy(src_hbm.at[0], idx_s.at[b], isems[b]).wait()
        pltpu.make_async_copy(dst_hbm.at[0], idx_d.at[b], isems[b]).wait()

    def _gather(b):
        pltpu.async_copy(feat_hbm.at[idx_s.at[b]], rows.at[b], gsems[b])

    def _wait_gather(b):
        pltpu.make_async_copy(feat_hbm.at[pl.ds(0, CHUNK)], rows.at[b],
                              gsems[b]).wait()

    def _scatter(b):
        pltpu.async_copy(rows.at[b], acc_sp.at[idx_d.at[b]], ssems[b],
                         add=True)

    def _wait_scatter(b):
        pltpu.make_async_copy(rows.at[b], acc_sp.at[pl.ds(0, CHUNK)],
                              ssems[b]).wait()

    def _hist(b):
        for j in range(CHUNK // L):
            idxv = idx_d[b, pl.ds(j * L, L)]
            rowi = lax.shift_right_logical(idxv, 7)
            coli = lax.bitwise_and(idxv, D - 1)
            plsc.addupdate_scatter(hist, [rowi, coli], ones16)

    pltpu.sync_copy(src_hbm.at[wid], idx_s.at[0])
    pltpu.sync_copy(dst_hbm.at[wid], idx_d.at[0])
    pltpu.sync_copy(src_hbm.at[wid + NW], idx_s.at[1])
    pltpu.sync_copy(dst_hbm.at[wid + NW], idx_d.at[1])
    _gather(0)
    _gather(1)

    def _pair(m, _):
        not_last = m < STEPS // 2 - 1
        _wait_gather(0)
        _scatter(0)
        _hist(0)
        _wait_gather(1)
        _scatter(1)
        _hist(1)

        @pl.when(not_last)
        def _():
            r2 = wid + (2 * m + 2) * NW
            _wait_scatter(0)
            _load_idx(0, r2)
            _wait_scatter(1)
            _load_idx(1, r2 + NW)
            _wait_idx(0)
            _gather(0)
            _wait_idx(1)
            _gather(1)
        return 0
    lax.fori_loop(0, STEPS // 2, _pair, 0)
    _wait_scatter(0)
    _wait_scatter(1)

    pltpu.sync_copy(hist, deg_sp.at[idrows.at[0]], add=True)
    plsc.subcore_barrier()

    def _flush(i, _):
        off = base + i * WB
        pltpu.sync_copy(acc_sp.at[pl.ds(off, WB)], rows.at[0])
        pltpu.sync_copy(rows.at[0], psum_hbm.at[cid, pl.ds(off, WB)])
        return 0
    lax.fori_loop(0, ROWS_PER_TILE // WB, _flush, 0)

    @pl.when(sid == 0)
    def _():
        pltpu.sync_copy(deg_sp, hist)
        pltpu.sync_copy(hist, pdeg_hbm.at[cid])


BLK = 640


def _tc_body(psum_ref, deg_ref, fdst_ref, ws_ref, wn_ref, b_ref, out_ref):
    neigh_sum = psum_ref[0] + psum_ref[1]
    deg = deg_ref[0] + deg_ref[1]
    h_neigh = neigh_sum / jnp.maximum(deg, 1.0)
    self_proj = lax.dot_general(fdst_ref[...], ws_ref[...],
                                (((1,), (1,)), ((), ())),
                                preferred_element_type=jnp.float32)
    neigh_proj = lax.dot_general(h_neigh, wn_ref[...],
                                 (((1,), (1,)), ((), ())),
                                 preferred_element_type=jnp.float32)
    out_ref[...] = self_proj + neigh_proj + b_ref[...]


_tc_combine = pl.pallas_call(
    _tc_body,
    grid=(N_PAD // BLK,),
    in_specs=[
        pl.BlockSpec((NC, BLK, D), lambda i: (0, i, 0)),
        pl.BlockSpec((NC, BLK, 1), lambda i: (0, i, 0)),
        pl.BlockSpec((BLK, D), lambda i: (i, 0)),
        pl.BlockSpec((D, D), lambda i: (0, 0)),
        pl.BlockSpec((D, D), lambda i: (0, 0)),
        pl.BlockSpec((1, D), lambda i: (0, 0)),
    ],
    out_specs=pl.BlockSpec((BLK, D), lambda i: (i, 0)),
    out_shape=jax.ShapeDtypeStruct((N_PAD, D), jnp.float32),
)


def kernel(feat_src, feat_dst, edge_index, W_self, b_self, W_neigh, b_neigh):
    pad = NROWS_PAD * CHUNK - E
    src = jnp.concatenate(
        [edge_index[0].astype(jnp.int32), jnp.zeros((pad,), jnp.int32)]
    ).reshape(NROWS_PAD, CHUNK)
    dst = jnp.concatenate(
        [edge_index[1].astype(jnp.int32), jnp.full((pad,), N_DST, jnp.int32)]
    ).reshape(NROWS_PAD, CHUNK)
    psum, pdeg = _sc_aggregate(feat_src, src, dst)
    deg_col = pdeg.reshape(NC, N_PAD, 1)
    bias = (b_self + b_neigh).reshape(1, D)
    rst = _tc_combine(psum, deg_col, feat_dst, W_self, W_neigh, bias)
    return rst[:N_DST]


# speedup vs baseline: 2.2121x; 2.2121x over previous
import functools

import jax
import jax.numpy as jnp
from jax import lax
from jax.experimental import pallas as pl
from jax.experimental.pallas import tpu as pltpu
from jax.experimental.pallas import tpu_sc as plsc

N_SRC = 10000
N_DST = 10000
E = 320000
D = 128

NC = 2
NS = 16
L = 16
NW = NC * NS
CHUNK = 128
NROWS = E // CHUNK
STEPS = 80
NROWS_PAD = NW * STEPS
N_PAD = 10240
HR = N_PAD // D
ROWS_PER_TILE = N_PAD // NS
WB = 128

_sc_mesh = plsc.VectorSubcoreMesh(
    core_axis_name="c", subcore_axis_name="s", num_cores=NC, num_subcores=NS)


@functools.partial(
    pl.kernel,
    out_type=(
        jax.ShapeDtypeStruct((NC, N_PAD, D), jnp.float32),
        jax.ShapeDtypeStruct((NC, HR, D), jnp.float32),
    ),
    mesh=_sc_mesh,
    compiler_params=pltpu.CompilerParams(
        use_tc_tiling_on_sc=False, needs_layout_passes=False),
    scratch_types=[
        pltpu.VMEM((2, CHUNK), jnp.int32),
        pltpu.VMEM((2, CHUNK), jnp.int32),
        pltpu.VMEM((2, CHUNK, D), jnp.float32),
        pltpu.VMEM((HR, D), jnp.float32),
        pltpu.VMEM((1, HR), jnp.int32),
        pltpu.VMEM_SHARED((N_PAD, D), jnp.float32),
        pltpu.VMEM_SHARED((HR, D), jnp.float32),
        pltpu.SemaphoreType.DMA,
        pltpu.SemaphoreType.DMA,
        pltpu.SemaphoreType.DMA,
        pltpu.SemaphoreType.DMA,
        pltpu.SemaphoreType.DMA,
        pltpu.SemaphoreType.DMA,
    ],
)
def _sc_aggregate(feat_hbm, src_hbm, dst_hbm, psum_hbm, pdeg_hbm,
                  idx_s, idx_d, rows, hist, idrows,
                  acc_sp, deg_sp, gsem0, gsem1, ssem0, ssem1, isem0, isem1):
    cid = lax.axis_index("c")
    sid = lax.axis_index("s")
    wid = sid * NC + cid
    base = sid * ROWS_PER_TILE
    ones16 = jnp.ones((L,), jnp.float32)

    def _fill_f(i, _):
        rows[0, i // (D // L), pl.ds((i % (D // L)) * L, L)] = (
            jnp.zeros((L,), jnp.float32))
        return 0
    lax.fori_loop(0, WB * (D // L), _fill_f, 0)

    def _fill_h(i, _):
        hist[i // (D // L), pl.ds((i % (D // L)) * L, L)] = (
            jnp.zeros((L,), jnp.float32))
        return 0
    lax.fori_loop(0, HR * (D // L), _fill_h, 0)

    for j in range(HR // L):
        idrows[0, pl.ds(j * L, L)] = lax.iota(jnp.int32, L) + (j * L)

    def _zinit(i, _):
        pltpu.sync_copy(rows.at[0], acc_sp.at[pl.ds(base + i * WB, WB)])
        return 0
    lax.fori_loop(0, ROWS_PER_TILE // WB, _zinit, 0)

    @pl.when(sid == 0)
    def _():
        pltpu.sync_copy(hist, deg_sp)
    plsc.subcore_barrier()

    gsems = (gsem0, gsem1)
    ssems = (ssem0, ssem1)
    isems = (isem0, isem1)

    def _load_idx(b, r):
        pltpu.async_copy(src_hbm.at[r], idx_s.at[b], isems[b])
        pltpu.async_copy(dst_hbm.at[r], idx_d.at[b], isems[b])

    def _wait_idx(b):
        pltpu.make_async_copy(src_hbm.at[0], idx_s.at[b], isems[b]).wait()
        pltpu.make_async_copy(dst_hbm.at[0], idx_d.at[b], isems[b]).wait()

    def _gather(b):
        pltpu.async_copy(feat_hbm.at[idx_s.at[b]], rows.at[b], gsems[b])

    def _wait_gather(b):
        pltpu.make_async_copy(feat_hbm.at[pl.ds(0, CHUNK)], rows.at[b],
                              gsems[b]).wait()

    def _scatter(b):
        pltpu.async_copy(rows.at[b], acc_sp.at[idx_d.at[b]], ssems[b],
                         add=True)

    def _wait_scatter(b):
        pltpu.make_async_copy(rows.at[b], acc_sp.at[pl.ds(0, CHUNK)],
                              ssems[b]).wait()

    def _hist(b):
        for j in range(CHUNK // L):
            idxv = idx_d[b, pl.ds(j * L, L)]
            rowi = lax.shift_right_logical(idxv, 7)
            coli = lax.bitwise_and(idxv, D - 1)
            plsc.addupdate_scatter(hist, [rowi, coli], ones16)

    pltpu.sync_copy(src_hbm.at[wid], idx_s.at[0])
    pltpu.sync_copy(dst_hbm.at[wid], idx_d.at[0])
    pltpu.sync_copy(src_hbm.at[wid + NW], idx_s.at[1])
    pltpu.sync_copy(dst_hbm.at[wid + NW], idx_d.at[1])
    _gather(0)
    _gather(1)

    def _pair(m, _):
        not_last = m < STEPS // 2 - 1
        _wait_gather(0)
        _scatter(0)
        _hist(0)
        _wait_gather(1)
        _scatter(1)
        _hist(1)

        @pl.when(not_last)
        def _():
            r2 = wid + (2 * m + 2) * NW
            _wait_scatter(0)
            _load_idx(0, r2)
            _wait_scatter(1)
            _load_idx(1, r2 + NW)
            _wait_idx(0)
            _gather(0)
            _wait_idx(1)
            _gather(1)
        return 0
    lax.fori_loop(0, STEPS // 2, _pair, 0)
    _wait_scatter(0)
    _wait_scatter(1)

    pltpu.sync_copy(hist, deg_sp.at[idrows.at[0]], add=True)
    plsc.subcore_barrier()

    def _flush(i, _):
        off = base + i * WB
        pltpu.sync_copy(acc_sp.at[pl.ds(off, WB)], rows.at[0])
        pltpu.sync_copy(rows.at[0], psum_hbm.at[cid, pl.ds(off, WB)])
        return 0
    lax.fori_loop(0, ROWS_PER_TILE // WB, _flush, 0)

    @pl.when(sid == 0)
    def _():
        pltpu.sync_copy(deg_sp, hist)
        pltpu.sync_copy(hist, pdeg_hbm.at[cid])


BLK = 640


def _tc_body(psum_ref, deg_ref, fdst_ref, ws_ref, wn_ref, b_ref, out_ref):
    neigh_sum = psum_ref[0] + psum_ref[1]
    deg = deg_ref[0] + deg_ref[1]
    h_neigh = neigh_sum / jnp.maximum(deg, 1.0)
    self_proj = lax.dot_general(fdst_ref[...], ws_ref[...],
                                (((1,), (1,)), ((), ())),
                                preferred_element_type=jnp.float32)
    neigh_proj = lax.dot_general(h_neigh, wn_ref[...],
                                 (((1,), (1,)), ((), ())),
                                 preferred_element_type=jnp.float32)
    out_ref[...] = self_proj + neigh_proj + b_ref[...]


_tc_combine = pl.pallas_call(
    _tc_body,
    grid=(N_PAD // BLK,),
    in_specs=[
        pl.BlockSpec((NC, BLK, D), lambda i: (0, i, 0)),
        pl.BlockSpec((NC, BLK, 1), lambda i: (0, i, 0)),
        pl.BlockSpec((BLK, D), lambda i: (i, 0)),
        pl.BlockSpec((D, D), lambda i: (0, 0)),
        pl.BlockSpec((D, D), lambda i: (0, 0)),
        pl.BlockSpec((1, D), lambda i: (0, 0)),
    ],
    out_specs=pl.BlockSpec((BLK, D), lambda i: (i, 0)),
    out_shape=jax.ShapeDtypeStruct((N_PAD, D), jnp.float32),
)


def kernel(feat_src, feat_dst, edge_index, W_self, b_self, W_neigh, b_neigh):
    pad = NROWS_PAD * CHUNK - E
    ar = jnp.arange(pad, dtype=jnp.int32)
    src = jnp.concatenate(
        [edge_index[0].astype(jnp.int32), ar % N_SRC]
    ).reshape(NROWS_PAD, CHUNK)
    dst = jnp.concatenate(
        [edge_index[1].astype(jnp.int32), N_DST + ar % (N_PAD - N_DST)]
    ).reshape(NROWS_PAD, CHUNK)
    psum, pdeg = _sc_aggregate(feat_src, src, dst)
    deg_col = pdeg.reshape(NC, N_PAD, 1)
    bias = (b_self + b_neigh).reshape(1, D)
    rst = _tc_combine(psum, deg_col, feat_dst, W_self, W_neigh, bias)
    return rst[:N_DST]


# speedup vs baseline: 2.2650x; 1.0239x over previous
import functools

import jax
import jax.numpy as jnp
from jax import lax
from jax.experimental import pallas as pl
from jax.experimental.pallas import tpu as pltpu
from jax.experimental.pallas import tpu_sc as plsc

N_SRC = 10000
N_DST = 10000
E = 320000
D = 128

NC = 2
NS = 16
L = 16
NW = NC * NS
CHUNK = 128
NROWS = E // CHUNK
STEPS = 80
NROWS_PAD = NW * STEPS
N_PAD = 10240
HR = N_PAD // D
ROWS_PER_TILE = N_PAD // NS
WB = 128

_sc_mesh = plsc.VectorSubcoreMesh(
    core_axis_name="c", subcore_axis_name="s", num_cores=NC, num_subcores=NS)


@functools.partial(
    pl.kernel,
    out_type=(
        jax.ShapeDtypeStruct((NC, N_PAD, D), jnp.float32),
        jax.ShapeDtypeStruct((NC, HR, D), jnp.float32),
    ),
    mesh=_sc_mesh,
    compiler_params=pltpu.CompilerParams(
        use_tc_tiling_on_sc=False, needs_layout_passes=False),
    scratch_types=[
        pltpu.VMEM((2, CHUNK), jnp.int32),
        pltpu.VMEM((2, CHUNK), jnp.int32),
        pltpu.VMEM((2, CHUNK, D), jnp.float32),
        pltpu.VMEM((HR, D), jnp.float32),
        pltpu.VMEM((1, HR), jnp.int32),
        pltpu.VMEM_SHARED((N_PAD, D), jnp.float32),
        pltpu.VMEM_SHARED((HR, D), jnp.float32),
        pltpu.SemaphoreType.DMA,
        pltpu.SemaphoreType.DMA,
        pltpu.SemaphoreType.DMA,
        pltpu.SemaphoreType.DMA,
        pltpu.SemaphoreType.DMA,
        pltpu.SemaphoreType.DMA,
    ],
)
def _sc_aggregate(feat_hbm, src_hbm, dst_hbm, psum_hbm, pdeg_hbm,
                  idx_s, idx_d, rows, hist, idrows,
                  acc_sp, deg_sp, gsem0, gsem1, ssem0, ssem1, isem0, isem1):
    cid = lax.axis_index("c")
    sid = lax.axis_index("s")
    wid = sid * NC + cid
    base = sid * ROWS_PER_TILE
    ones16 = jnp.ones((L,), jnp.float32)

    def _fill_f(i, _):
        rows[0, i // (D // L), pl.ds((i % (D // L)) * L, L)] = (
            jnp.zeros((L,), jnp.float32))
        return 0
    lax.fori_loop(0, WB * (D // L), _fill_f, 0)

    def _fill_h(i, _):
        hist[i // (D // L), pl.ds((i % (D // L)) * L, L)] = (
            jnp.zeros((L,), jnp.float32))
        return 0
    lax.fori_loop(0, HR * (D // L), _fill_h, 0)

    for j in range(HR // L):
        idrows[0, pl.ds(j * L, L)] = lax.iota(jnp.int32, L) + (j * L)

    def _zinit(i, _):
        pltpu.sync_copy(rows.at[0], acc_sp.at[pl.ds(base + i * WB, WB)])
        return 0
    lax.fori_loop(0, ROWS_PER_TILE // WB, _zinit, 0)

    @pl.when(sid == 0)
    def _():
        pltpu.sync_copy(hist, deg_sp)
    plsc.subcore_barrier()

    gsems = (gsem0, gsem1)
    ssems = (ssem0, ssem1)
    isems = (isem0, isem1)

    def _load_idx(b, r):
        pltpu.async_copy(src_hbm.at[r], idx_s.at[b], isems[b])
        pltpu.async_copy(dst_hbm.at[r], idx_d.at[b], isems[b])

    def _wait_idx(b):
        pltpu.make_async_copy(src_hbm.at[0], idx_s.at[b], isems[b]).wait()
        pltpu.make_async_copy(dst_hbm.at[0], idx_d.at[b], isems[b]).wait()

    def _gather(b):
        pltpu.async_copy(feat_hbm.at[idx_s.at[b]], rows.at[b], gsems[b])

    def _wait_gather(b):
        pltpu.make_async_copy(feat_hbm.at[pl.ds(0, CHUNK)], rows.at[b],
                              gsems[b]).wait()

    def _scatter(b):
        pltpu.async_copy(rows.at[b], acc_sp.at[idx_d.at[b]], ssems[b],
                         add=True)

    def _wait_scatter(b):
        pltpu.make_async_copy(rows.at[b], acc_sp.at[pl.ds(0, CHUNK)],
                              ssems[b]).wait()

    def _hist(b):
        for j in range(CHUNK // L):
            idxv = idx_d[b, pl.ds(j * L, L)]
            rowi = lax.shift_right_logical(idxv, 7)
            coli = lax.bitwise_and(idxv, D - 1)
            plsc.addupdate_scatter(hist, [rowi, coli], ones16)

    pltpu.sync_copy(src_hbm.at[wid], idx_s.at[0])
    pltpu.sync_copy(dst_hbm.at[wid], idx_d.at[0])
    pltpu.sync_copy(src_hbm.at[wid + NW], idx_s.at[1])
    pltpu.sync_copy(dst_hbm.at[wid + NW], idx_d.at[1])
    _gather(0)
    _gather(1)

    def _pair(m, _):
        not_last = m < STEPS // 2 - 1
        _wait_gather(0)
        _scatter(0)
        _hist(0)
        _wait_gather(1)
        _scatter(1)
        _hist(1)

        @pl.when(not_last)
        def _():
            r2 = wid + (2 * m + 2) * NW
            _wait_scatter(0)
            _load_idx(0, r2)
            _wait_scatter(1)
            _load_idx(1, r2 + NW)
            _wait_idx(0)
            _gather(0)
            _wait_idx(1)
            _gather(1)
        return 0
    lax.fori_loop(0, STEPS // 2, _pair, 0)
    _wait_scatter(0)
    _wait_scatter(1)

    pltpu.sync_copy(hist, deg_sp.at[idrows.at[0]], add=True)
    plsc.subcore_barrier()

    def _flush(i, _):
        off = base + i * WB
        pltpu.sync_copy(acc_sp.at[pl.ds(off, WB)], rows.at[0])
        pltpu.sync_copy(rows.at[0], psum_hbm.at[cid, pl.ds(off, WB)])
        return 0
    lax.fori_loop(0, ROWS_PER_TILE // WB, _flush, 0)

    @pl.when(sid == 0)
    def _():
        pltpu.sync_copy(deg_sp, hist)
        pltpu.sync_copy(hist, pdeg_hbm.at[cid])


BLK = 640


def _tc_body(psum_ref, deg_ref, fdst_ref, ws_ref, wn_ref, b_ref, out_ref):
    neigh_sum = psum_ref[0] + psum_ref[1]
    deg = deg_ref[0] + deg_ref[1]
    h_neigh = neigh_sum / jnp.maximum(deg, 1.0)
    self_proj = lax.dot_general(fdst_ref[...], ws_ref[...],
                                (((1,), (1,)), ((), ())),
                                preferred_element_type=jnp.float32)
    neigh_proj = lax.dot_general(h_neigh, wn_ref[...],
                                 (((1,), (1,)), ((), ())),
                                 preferred_element_type=jnp.float32)
    out_ref[...] = self_proj + neigh_proj + b_ref[...]


_tc_combine = pl.pallas_call(
    _tc_body,
    grid=(N_PAD // BLK,),
    in_specs=[
        pl.BlockSpec((NC, BLK, D), lambda i: (0, i, 0)),
        pl.BlockSpec((NC, BLK, 1), lambda i: (0, i, 0)),
        pl.BlockSpec((BLK, D), lambda i: (i, 0)),
        pl.BlockSpec((D, D), lambda i: (0, 0)),
        pl.BlockSpec((D, D), lambda i: (0, 0)),
        pl.BlockSpec((1, D), lambda i: (0, 0)),
    ],
    out_specs=pl.BlockSpec((BLK, D), lambda i: (i, 0)),
    out_shape=jax.ShapeDtypeStruct((N_DST, D), jnp.float32),
)


def kernel(feat_src, feat_dst, edge_index, W_self, b_self, W_neigh, b_neigh):
    pad = NROWS_PAD * CHUNK - E
    ar = jnp.arange(pad, dtype=jnp.int32)
    src = jnp.concatenate(
        [edge_index[0].astype(jnp.int32), ar % N_SRC]
    ).reshape(NROWS_PAD, CHUNK)
    dst = jnp.concatenate(
        [edge_index[1].astype(jnp.int32), N_DST + ar % (N_PAD - N_DST)]
    ).reshape(NROWS_PAD, CHUNK)
    psum, pdeg = _sc_aggregate(feat_src, src, dst)
    deg_col = pdeg.reshape(NC, N_PAD, 1)
    bias = (b_self + b_neigh).reshape(1, D)
    return _tc_combine(psum, deg_col, feat_dst, W_self, W_neigh, bias)


# speedup vs baseline: 2.2725x; 1.0033x over previous
import functools

import jax
import jax.numpy as jnp
from jax import lax
from jax.experimental import pallas as pl
from jax.experimental.pallas import tpu as pltpu
from jax.experimental.pallas import tpu_sc as plsc

N_SRC = 10000
N_DST = 10000
E = 320000
D = 128

NC = 2
NS = 16
L = 16
NW = NC * NS
CHUNK = 128
NROWS = E // CHUNK
STEPS = 80
NROWS_PAD = NW * STEPS
N_PAD = 10240
HR = N_PAD // D
ROWS_PER_TILE = N_PAD // NS
WB = 128

_sc_mesh = plsc.VectorSubcoreMesh(
    core_axis_name="c", subcore_axis_name="s", num_cores=NC, num_subcores=NS)


@functools.partial(
    pl.kernel,
    out_type=(
        jax.ShapeDtypeStruct((NC, N_PAD, D), jnp.float32),
        jax.ShapeDtypeStruct((NC, HR, D), jnp.float32),
    ),
    mesh=_sc_mesh,
    compiler_params=pltpu.CompilerParams(
        use_tc_tiling_on_sc=False, needs_layout_passes=False),
    scratch_types=[
        pltpu.VMEM((2, CHUNK), jnp.int32),
        pltpu.VMEM((2, CHUNK), jnp.int32),
        pltpu.VMEM((2, CHUNK, D), jnp.float32),
        pltpu.VMEM((HR, D), jnp.float32),
        pltpu.VMEM((1, HR), jnp.int32),
        pltpu.VMEM_SHARED((N_PAD, D), jnp.float32),
        pltpu.VMEM_SHARED((HR, D), jnp.float32),
        pltpu.SemaphoreType.DMA,
        pltpu.SemaphoreType.DMA,
        pltpu.SemaphoreType.DMA,
        pltpu.SemaphoreType.DMA,
        pltpu.SemaphoreType.DMA,
        pltpu.SemaphoreType.DMA,
    ],
)
def _sc_aggregate(feat_hbm, src_hbm, dst_hbm, psum_hbm, pdeg_hbm,
                  idx_s, idx_d, rows, hist, idrows,
                  acc_sp, deg_sp, gsem0, gsem1, ssem0, ssem1, isem0, isem1):
    cid = lax.axis_index("c")
    sid = lax.axis_index("s")
    wid = sid * NC + cid
    base = sid * ROWS_PER_TILE
    ones16 = jnp.ones((L,), jnp.float32)

    def _fill_f(i, _):
        rows[0, i // (D // L), pl.ds((i % (D // L)) * L, L)] = (
            jnp.zeros((L,), jnp.float32))
        return 0
    lax.fori_loop(0, WB * (D // L), _fill_f, 0)

    def _fill_h(i, _):
        hist[i // (D // L), pl.ds((i % (D // L)) * L, L)] = (
            jnp.zeros((L,), jnp.float32))
        return 0
    lax.fori_loop(0, HR * (D // L), _fill_h, 0)

    for j in range(HR // L):
        idrows[0, pl.ds(j * L, L)] = lax.iota(jnp.int32, L) + (j * L)

    def _zinit(i, _):
        pltpu.sync_copy(rows.at[0], acc_sp.at[pl.ds(base + i * WB, WB)])
        return 0
    lax.fori_loop(0, ROWS_PER_TILE // WB, _zinit, 0)

    @pl.when(sid == 0)
    def _():
        pltpu.sync_copy(hist, deg_sp)
    plsc.subcore_barrier()

    gsems = (gsem0, gsem1)
    ssems = (ssem0, ssem1)
    isems = (isem0, isem1)

    def _load_idx(b, r):
        pltpu.async_copy(src_hbm.at[r], idx_s.at[b], isems[b])
        pltpu.async_copy(dst_hbm.at[r], idx_d.at[b], isems[b])

    def _wait_idx(b):
        pltpu.make_async_copy(src_hbm.at[0], idx_s.at[b], isems[b]).wait()
        pltpu.make_async_copy(dst_hbm.at[0], idx_d.at[b], isems[b]).wait()

    def _gather(b):
        pltpu.async_copy(feat_hbm.at[idx_s.at[b]], rows.at[b], gsems[b])

    def _wait_gather(b):
        pltpu.make_async_copy(feat_hbm.at[pl.ds(0, CHUNK)], rows.at[b],
                              gsems[b]).wait()

    def _scatter(b):
        pltpu.async_copy(rows.at[b], acc_sp.at[idx_d.at[b]], ssems[b],
                         add=True)

    def _wait_scatter(b):
        pltpu.make_async_copy(rows.at[b], acc_sp.at[pl.ds(0, CHUNK)],
                              ssems[b]).wait()

    def _hist(b):
        for j in range(CHUNK // L):
            idxv = idx_d[b, pl.ds(j * L, L)]
            rowi = lax.shift_right_logical(idxv, 7)
            coli = lax.bitwise_and(idxv, D - 1)
            plsc.addupdate_scatter(hist, [rowi, coli], ones16)

    pltpu.sync_copy(src_hbm.at[wid], idx_s.at[0])
    pltpu.sync_copy(dst_hbm.at[wid], idx_d.at[0])
    pltpu.sync_copy(src_hbm.at[wid + NW], idx_s.at[1])
    pltpu.sync_copy(dst_hbm.at[wid + NW], idx_d.at[1])
    _gather(0)
    _gather(1)

    def _pair(m, _):
        not_last = m < STEPS // 2 - 1
        _wait_gather(0)
        _scatter(0)
        _hist(0)
        _wait_gather(1)
        _scatter(1)
        _hist(1)

        @pl.when(not_last)
        def _():
            r2 = wid + (2 * m + 2) * NW
            _wait_scatter(0)
            _load_idx(0, r2)
            _wait_scatter(1)
            _load_idx(1, r2 + NW)
            _wait_idx(0)
            _gather(0)
            _wait_idx(1)
            _gather(1)
        return 0
    lax.fori_loop(0, STEPS // 2, _pair, 0)
    _wait_scatter(0)
    _wait_scatter(1)

    pltpu.sync_copy(hist, deg_sp.at[idrows.at[0]], add=True)
    plsc.subcore_barrier()

    def _flush(i, _):
        off = base + i * WB
        pltpu.sync_copy(acc_sp.at[pl.ds(off, WB)],
                        psum_hbm.at[cid, pl.ds(off, WB)])
        return 0
    lax.fori_loop(0, ROWS_PER_TILE // WB, _flush, 0)

    @pl.when(sid == 0)
    def _():
        pltpu.sync_copy(deg_sp, hist)
        pltpu.sync_copy(hist, pdeg_hbm.at[cid])


BLK = 640


def _tc_body(psum_ref, deg_ref, fdst_ref, ws_ref, wn_ref, b_ref, out_ref):
    neigh_sum = psum_ref[0] + psum_ref[1]
    deg = deg_ref[0] + deg_ref[1]
    h_neigh = neigh_sum / jnp.maximum(deg, 1.0)
    self_proj = lax.dot_general(fdst_ref[...], ws_ref[...],
                                (((1,), (1,)), ((), ())),
                                preferred_element_type=jnp.float32)
    neigh_proj = lax.dot_general(h_neigh, wn_ref[...],
                                 (((1,), (1,)), ((), ())),
                                 preferred_element_type=jnp.float32)
    out_ref[...] = self_proj + neigh_proj + b_ref[...]


_tc_combine = pl.pallas_call(
    _tc_body,
    grid=(N_PAD // BLK,),
    in_specs=[
        pl.BlockSpec((NC, BLK, D), lambda i: (0, i, 0)),
        pl.BlockSpec((NC, BLK, 1), lambda i: (0, i, 0)),
        pl.BlockSpec((BLK, D), lambda i: (i, 0)),
        pl.BlockSpec((D, D), lambda i: (0, 0)),
        pl.BlockSpec((D, D), lambda i: (0, 0)),
        pl.BlockSpec((1, D), lambda i: (0, 0)),
    ],
    out_specs=pl.BlockSpec((BLK, D), lambda i: (i, 0)),
    out_shape=jax.ShapeDtypeStruct((N_DST, D), jnp.float32),
)


def kernel(feat_src, feat_dst, edge_index, W_self, b_self, W_neigh, b_neigh):
    pad = NROWS_PAD * CHUNK - E
    ar = jnp.arange(pad, dtype=jnp.int32)
    src = jnp.concatenate(
        [edge_index[0].astype(jnp.int32), ar % N_SRC]
    ).reshape(NROWS_PAD, CHUNK)
    dst = jnp.concatenate(
        [edge_index[1].astype(jnp.int32), N_DST + ar % (N_PAD - N_DST)]
    ).reshape(NROWS_PAD, CHUNK)
    psum, pdeg = _sc_aggregate(feat_src, src, dst)
    deg_col = pdeg.reshape(NC, N_PAD, 1)
    bias = (b_self + b_neigh).reshape(1, D)
    return _tc_combine(psum, deg_col, feat_dst, W_self, W_neigh, bias)


# speedup vs baseline: 2.4698x; 1.0868x over previous
import functools

import jax
import jax.numpy as jnp
from jax import lax
from jax.experimental import pallas as pl
from jax.experimental.pallas import tpu as pltpu
from jax.experimental.pallas import tpu_sc as plsc

N_SRC = 10000
N_DST = 10000
E = 320000
D = 128

NC = 2
NS = 16
L = 16
NW = NC * NS
CHUNK = 128
NROWS = E // CHUNK
STEPS = 80
NROWS_PAD = NW * STEPS
N_PAD = 10240
HR = N_PAD // D
ROWS_PER_TILE = N_PAD // NS
WB = 128

_sc_mesh = plsc.VectorSubcoreMesh(
    core_axis_name="c", subcore_axis_name="s", num_cores=NC, num_subcores=NS)


@functools.partial(
    pl.kernel,
    out_type=(
        jax.ShapeDtypeStruct((NC, N_PAD, D), jnp.float32),
        jax.ShapeDtypeStruct((NC, HR, D), jnp.float32),
    ),
    mesh=_sc_mesh,
    compiler_params=pltpu.CompilerParams(
        use_tc_tiling_on_sc=False, needs_layout_passes=False),
    scratch_types=[
        pltpu.VMEM((2, CHUNK), jnp.int32),
        pltpu.VMEM((2, CHUNK), jnp.int32),
        pltpu.VMEM((2, CHUNK), jnp.int32),
        pltpu.VMEM((2, CHUNK), jnp.int32),
        pltpu.VMEM((2, CHUNK, D), jnp.float32),
        pltpu.VMEM((HR, D), jnp.float32),
        pltpu.VMEM((1, HR), jnp.int32),
        pltpu.VMEM_SHARED((N_PAD, D), jnp.float32),
        pltpu.VMEM_SHARED((HR, D), jnp.float32),
        pltpu.SemaphoreType.DMA,
        pltpu.SemaphoreType.DMA,
        pltpu.SemaphoreType.DMA,
        pltpu.SemaphoreType.DMA,
        pltpu.SemaphoreType.DMA,
        pltpu.SemaphoreType.DMA,
    ],
)
def _sc_aggregate(feat_hbm, src_hbm, dst_hbm, psum_hbm, pdeg_hbm,
                  idx_s, idx_d, nidx_s, nidx_d, rows, hist, idrows,
                  acc_sp, deg_sp, gsem0, gsem1, ssem0, ssem1, isem0, isem1):
    cid = lax.axis_index("c")
    sid = lax.axis_index("s")
    wid = sid * NC + cid
    base = sid * ROWS_PER_TILE
    ones16 = jnp.ones((L,), jnp.float32)

    def _fill_f(i, _):
        rows[0, i // (D // L), pl.ds((i % (D // L)) * L, L)] = (
            jnp.zeros((L,), jnp.float32))
        return 0
    lax.fori_loop(0, WB * (D // L), _fill_f, 0)

    def _fill_h(i, _):
        hist[i // (D // L), pl.ds((i % (D // L)) * L, L)] = (
            jnp.zeros((L,), jnp.float32))
        return 0
    lax.fori_loop(0, HR * (D // L), _fill_h, 0)

    for j in range(HR // L):
        idrows[0, pl.ds(j * L, L)] = lax.iota(jnp.int32, L) + (j * L)

    def _zinit(i, _):
        pltpu.sync_copy(rows.at[0], acc_sp.at[pl.ds(base + i * WB, WB)])
        return 0
    lax.fori_loop(0, ROWS_PER_TILE // WB, _zinit, 0)

    @pl.when(sid == 0)
    def _():
        pltpu.sync_copy(hist, deg_sp)
    plsc.subcore_barrier()

    gsems = (gsem0, gsem1)
    ssems = (ssem0, ssem1)
    isems = (isem0, isem1)

    def _load_idx(b, r):
        pltpu.async_copy(src_hbm.at[r], idx_s.at[b], isems[b])
        pltpu.async_copy(dst_hbm.at[r], idx_d.at[b], isems[b])

    def _wait_idx(b):
        pltpu.make_async_copy(src_hbm.at[0], idx_s.at[b], isems[b]).wait()
        pltpu.make_async_copy(dst_hbm.at[0], idx_d.at[b], isems[b]).wait()

    def _gather(b):
        pltpu.async_copy(feat_hbm.at[idx_s.at[b]], rows.at[b], gsems[b])

    def _wait_gather(b):
        pltpu.make_async_copy(feat_hbm.at[pl.ds(0, CHUNK)], rows.at[b],
                              gsems[b]).wait()

    def _scatter(b):
        pltpu.async_copy(rows.at[b], acc_sp.at[idx_d.at[b]], ssems[b],
                         add=True)

    def _wait_scatter(b):
        pltpu.make_async_copy(rows.at[b], acc_sp.at[pl.ds(0, CHUNK)],
                              ssems[b]).wait()

    def _hist(b):
        for j in range(CHUNK // L):
            idxv = idx_d[b, pl.ds(j * L, L)]
            rowi = lax.shift_right_logical(idxv, 7)
            coli = lax.bitwise_and(idxv, D - 1)
            plsc.addupdate_scatter(hist, [rowi, coli], ones16)

    pltpu.sync_copy(src_hbm.at[wid], idx_s.at[0])
    pltpu.sync_copy(dst_hbm.at[wid], idx_d.at[0])
    pltpu.sync_copy(src_hbm.at[wid + NW], idx_s.at[1])
    pltpu.sync_copy(dst_hbm.at[wid + NW], idx_d.at[1])
    _gather(0)
    _gather(1)

    def _pair(m, _):
        not_last = m < STEPS // 2 - 1

        @pl.when(not_last)
        def _():
            r2 = wid + (2 * m + 2) * NW
            pltpu.async_copy(src_hbm.at[r2], nidx_s.at[0], isem0)
            pltpu.async_copy(dst_hbm.at[r2], nidx_d.at[0], isem0)
            pltpu.async_copy(src_hbm.at[r2 + NW], nidx_s.at[1], isem0)
            pltpu.async_copy(dst_hbm.at[r2 + NW], nidx_d.at[1], isem0)

        _wait_gather(0)
        _scatter(0)
        _hist(0)
        _wait_gather(1)
        _scatter(1)
        _hist(1)

        @pl.when(not_last)
        def _():
            for b in range(2):
                pltpu.make_async_copy(src_hbm.at[0], nidx_s.at[b],
                                      isem0).wait()
                pltpu.make_async_copy(dst_hbm.at[0], nidx_d.at[b],
                                      isem0).wait()
            _wait_scatter(0)
            for j in range(CHUNK // L):
                idx_s[0, pl.ds(j * L, L)] = nidx_s[0, pl.ds(j * L, L)]
                idx_d[0, pl.ds(j * L, L)] = nidx_d[0, pl.ds(j * L, L)]
            _gather(0)
            _wait_scatter(1)
            for j in range(CHUNK // L):
                idx_s[1, pl.ds(j * L, L)] = nidx_s[1, pl.ds(j * L, L)]
                idx_d[1, pl.ds(j * L, L)] = nidx_d[1, pl.ds(j * L, L)]
            _gather(1)
        return 0
    lax.fori_loop(0, STEPS // 2, _pair, 0)
    _wait_scatter(0)
    _wait_scatter(1)

    pltpu.sync_copy(hist, deg_sp.at[idrows.at[0]], add=True)
    plsc.subcore_barrier()

    def _flush(i, _):
        off = base + i * WB
        pltpu.sync_copy(acc_sp.at[pl.ds(off, WB)],
                        psum_hbm.at[cid, pl.ds(off, WB)])
        return 0
    lax.fori_loop(0, ROWS_PER_TILE // WB, _flush, 0)

    @pl.when(sid == 0)
    def _():
        pltpu.sync_copy(deg_sp, hist)
        pltpu.sync_copy(hist, pdeg_hbm.at[cid])


BLK = 640


def _tc_body(psum_ref, deg_ref, fdst_ref, ws_ref, wn_ref, b_ref, out_ref):
    neigh_sum = psum_ref[0] + psum_ref[1]
    deg = deg_ref[0] + deg_ref[1]
    h_neigh = neigh_sum / jnp.maximum(deg, 1.0)
    self_proj = lax.dot_general(fdst_ref[...], ws_ref[...],
                                (((1,), (1,)), ((), ())),
                                preferred_element_type=jnp.float32)
    neigh_proj = lax.dot_general(h_neigh, wn_ref[...],
                                 (((1,), (1,)), ((), ())),
                                 preferred_element_type=jnp.float32)
    out_ref[...] = self_proj + neigh_proj + b_ref[...]


_tc_combine = pl.pallas_call(
    _tc_body,
    grid=(N_PAD // BLK,),
    in_specs=[
        pl.BlockSpec((NC, BLK, D), lambda i: (0, i, 0)),
        pl.BlockSpec((NC, BLK, 1), lambda i: (0, i, 0)),
        pl.BlockSpec((BLK, D), lambda i: (i, 0)),
        pl.BlockSpec((D, D), lambda i: (0, 0)),
        pl.BlockSpec((D, D), lambda i: (0, 0)),
        pl.BlockSpec((1, D), lambda i: (0, 0)),
    ],
    out_specs=pl.BlockSpec((BLK, D), lambda i: (i, 0)),
    out_shape=jax.ShapeDtypeStruct((N_DST, D), jnp.float32),
)


def kernel(feat_src, feat_dst, edge_index, W_self, b_self, W_neigh, b_neigh):
    pad = NROWS_PAD * CHUNK - E
    ar = jnp.arange(pad, dtype=jnp.int32)
    src = jnp.concatenate(
        [edge_index[0].astype(jnp.int32), ar % N_SRC]
    ).reshape(NROWS_PAD, CHUNK)
    dst = jnp.concatenate(
        [edge_index[1].astype(jnp.int32), N_DST + ar % (N_PAD - N_DST)]
    ).reshape(NROWS_PAD, CHUNK)
    psum, pdeg = _sc_aggregate(feat_src, src, dst)
    deg_col = pdeg.reshape(NC, N_PAD, 1)
    bias = (b_self + b_neigh).reshape(1, D)
    return _tc_combine(psum, deg_col, feat_dst, W_self, W_neigh, bias)
